# baseline (speedup 1.0000x reference)
import functools

import jax
import jax.numpy as jnp
from jax import lax
from jax.experimental import pallas as pl
from jax.experimental.pallas import tpu as pltpu
from jax.experimental.pallas import tpu_sc as plsc


def _tc_body(v_ref, m_ref, o_ref, off_ref, val_ref, *, S, d):
    i = pl.program_id(0)
    vb = v_ref[0]
    w = 1.0 - m_ref[0]
    cols = lax.broadcasted_iota(jnp.int32, (d, S), 1)
    values = jnp.max(vb, axis=1, keepdims=True)
    idx = jnp.min(jnp.where(vb == values, cols, S), axis=1,
                  keepdims=True)
    v_cls = vb[:, 0:1]
    out = vb * w
    o_ref[0] = jnp.where(cols == 0, values * w[0:1, 0:1], out)
    w_at = jnp.min(jnp.where(cols == idx, jnp.broadcast_to(w, (d, S)), 2.0),
                   axis=1, keepdims=True)
    chan = lax.broadcasted_iota(jnp.int32, (d, 1), 0)
    off = (i * d + chan) * S + idx
    off_ref[...] = off.reshape(1, 1, d)
    val_ref[...] = (v_cls * w_at).reshape(1, 1, d)


def kernel(q, k, v, attn_mask):
    del q, k
    B, H, S, d = v.shape
    N = B * H
    vt = jnp.swapaxes(v, 2, 3).reshape(N, d, S)
    mf = attn_mask.astype(jnp.float32).reshape(N, 1, S)
    big, off, vals = pl.pallas_call(
        functools.partial(_tc_body, S=S, d=d),
        grid=(N,),
        in_specs=[
            pl.BlockSpec((1, d, S), lambda i: (i, 0, 0)),
            pl.BlockSpec((1, 1, S), lambda i: (i, 0, 0)),
        ],
        out_specs=[
            pl.BlockSpec((1, d, S), lambda i: (i, 0, 0)),
            pl.BlockSpec((1, 1, d), lambda i: (i, 0, 0)),
            pl.BlockSpec((1, 1, d), lambda i: (i, 0, 0)),
        ],
        out_shape=[
            jax.ShapeDtypeStruct((N, d, S), v.dtype),
            jax.ShapeDtypeStruct((N, 1, d), jnp.int32),
            jax.ShapeDtypeStruct((N, 1, d), jnp.float32),
        ],
    )(vt, mf)

    info = plsc.get_sparse_core_info()
    nw = info.num_cores * info.num_subcores
    per = (N * d) // nw
    mesh = plsc.VectorSubcoreMesh(core_axis_name="c", subcore_axis_name="s")

    @functools.partial(
        pl.kernel, mesh=mesh,
        scratch_types=[
            pltpu.VMEM((per,), jnp.int32),
            pltpu.VMEM((per,), jnp.float32),
            pltpu.SemaphoreType.DMA,
        ],
    )
    def _sc_scatter(off_hbm, val_hbm, out_flat, idx_v, val_v, sem):
        wid = lax.axis_index("s") * info.num_cores + lax.axis_index("c")
        base = wid * per
        pltpu.sync_copy(off_hbm.at[pl.ds(base, per)], idx_v)
        pltpu.sync_copy(val_hbm.at[pl.ds(base, per)], val_v)
        pltpu.async_copy(val_v, out_flat.at[idx_v], sem).wait()

    big_ref = jax.new_ref(big.reshape(N * d * S))
    _sc_scatter(off.reshape(N * d), vals.reshape(N * d), big_ref)
    out = big_ref[...].reshape(B, H, d, S)
    return jnp.swapaxes(out, 2, 3)


# speedup vs baseline: 3.2100x; 3.2100x over previous
import functools

import jax
import jax.numpy as jnp
from jax.experimental import pallas as pl


def _swd7_body(v_ref, m_ref, o_ref, *, S, d):
    vb = v_ref[0, 0]
    w = 1.0 - m_ref[0, 0]
    cols = jax.lax.broadcasted_iota(jnp.int32, (d, S), 1)
    values = jnp.max(vb, axis=1, keepdims=True)
    idx = jnp.min(jnp.where(vb == values, cols, S), axis=1,
                  keepdims=True)
    v_cls = vb[:, 0:1]
    out = jnp.where(cols == idx, v_cls, vb)
    o_ref[0, 0] = out * w
    o_ref[0, 0, :, 0:1] = values * w[0:1, 0:1]


def kernel(q, k, v, attn_mask):
    del q, k
    B, H, S, d = v.shape
    vt = jnp.swapaxes(v, 2, 3)
    mf = attn_mask.astype(jnp.float32)
    out = pl.pallas_call(
        functools.partial(_swd7_body, S=S, d=d),
        grid=(B, H),
        in_specs=[
            pl.BlockSpec((1, 1, d, S), lambda b, h: (b, h, 0, 0)),
            pl.BlockSpec((1, 1, 1, S), lambda b, h: (b, h, 0, 0)),
        ],
        out_specs=pl.BlockSpec((1, 1, d, S), lambda b, h: (b, h, 0, 0)),
        out_shape=jax.ShapeDtypeStruct((B, H, d, S), v.dtype),
    )(vt, mf)
    return jnp.swapaxes(out, 2, 3)
